# baseline (speedup 1.0000x reference)
import functools

import jax
import jax.numpy as jnp
from jax.experimental import pallas as pl
from jax.experimental.pallas import tpu as pltpu

_N = 10000
_T = 21
_F_IN = 128
_HID = 32
_BN = 384
_SCALE = 1.0 / (1.0 + 1e-5) ** 0.5


def _pack_taps_t(p):
    w1, b1, w2, b2, w3, b3 = p
    taps = [
        jnp.concatenate([w1[:, :, 0, k], w2[:, :, 0, k], w3[:, :, 0, k]], axis=0)
        for k in range(3)
    ]
    b = jnp.concatenate([b1, b2, b3]).reshape(3 * _HID, 1)
    return taps, b


def _pack_stacked_t(p):
    taps, b = _pack_taps_t(p)
    return jnp.concatenate(taps, axis=1), b


def _glu_t(Y):
    P = Y[0:32, :]
    Q = Y[32:64, :]
    R = Y[64:96, :]
    return jax.nn.relu(P * jax.nn.sigmoid(Q) + R)


def _tap_stack(H, t_out):
    L = t_out * _BN
    return jnp.concatenate(
        [H[:, 0:L], H[:, _BN:_BN + L], H[:, 2 * _BN:2 * _BN + L]], axis=0)


def _stgcn_block(x_ref, mask_ref, w1k0_ref, w1k1_ref, w1k2_ref, b1_ref,
                 wa_ref, ba_ref, w2_ref, b2_ref, w3_ref, b3_ref, wb_ref,
                 bb_ref, w4_ref, b4_ref, lw_ref, lb_ref, out_ref, acc_ref,
                 *, nblocks):
    i = pl.program_id(0)

    xb = x_ref[...].astype(jnp.bfloat16)
    X3 = jnp.transpose(xb, (0, 2, 1))
    xT = jnp.concatenate([X3[t] for t in range(_T)], axis=1)

    dot = functools.partial(jnp.dot, preferred_element_type=jnp.float32)
    A0 = dot(w1k0_ref[...], xT)
    A1 = dot(w1k1_ref[...], xT)
    A2 = dot(w1k2_ref[...], xT)
    L1 = 19 * _BN
    Y1 = (A0[:, 0:L1] + A1[:, _BN:_BN + L1] + A2[:, 2 * _BN:2 * _BN + L1]
          + b1_ref[...])
    H1 = _glu_t(Y1)
    Tc = jax.nn.relu(dot(wa_ref[...], H1) + ba_ref[...])
    H2 = _glu_t(dot(w2_ref[...], _tap_stack(Tc, 17)) + b2_ref[...]) * _SCALE
    H3 = _glu_t(dot(w3_ref[...], _tap_stack(H2, 15)) + b3_ref[...])
    Tc2 = jax.nn.relu(dot(wb_ref[...], H3) + bb_ref[...])
    H4 = _glu_t(dot(w4_ref[...], _tap_stack(Tc2, 13)) + b4_ref[...])

    mask = jnp.concatenate([mask_ref[0]] * 13, axis=1)
    H4 = jnp.where(mask > 0, H4, 0.0)
    part = jnp.sum(H4, axis=0, keepdims=True)

    @pl.when(i == 0)
    def _init():
        acc_ref[...] = jnp.zeros_like(acc_ref)

    acc_ref[...] += part

    @pl.when(i == nblocks - 1)
    def _finish():
        acc = acc_ref[...]
        a13 = jnp.concatenate(
            [acc[:, t * _BN:(t + 1) * _BN] for t in range(13)], axis=0)
        s = jnp.sum(a13, axis=1, keepdims=True)
        out = jnp.sum(s * lw_ref[...], axis=0, keepdims=True)
        out_ref[...] = out * (_SCALE / (_N * _HID)) + lb_ref[...]


def kernel(x, edge_index, edge_weight, tc1a, cheb_a, tc2a, tc1b, cheb_b, tc2b,
           lin_w, lin_b):
    del edge_index, edge_weight
    (W1k0, W1k1, W1k2), B1 = _pack_taps_t(tc1a)
    W1k0, W1k1, W1k2 = (w.astype(jnp.bfloat16) for w in (W1k0, W1k1, W1k2))
    W2, B2 = _pack_stacked_t(tc2a)
    W3, B3 = _pack_stacked_t(tc1b)
    W4, B4 = _pack_stacked_t(tc2b)
    Wa, ba = cheb_a
    Wb, bb = cheb_b
    Wa, Wb = Wa.T, Wb.T
    ba = ba.reshape(_HID, 1)
    bb = bb.reshape(_HID, 1)
    lb = lin_b.reshape(1, -1)

    nblocks = -(-_N // _BN)
    mask = (jnp.arange(nblocks * _BN, dtype=jnp.int32) < _N)
    mask = mask.astype(jnp.float32).reshape(nblocks, 1, _BN)

    full = lambda a: pl.BlockSpec(a.shape, lambda *_: tuple(0 for _ in a.shape))
    out = pl.pallas_call(
        functools.partial(_stgcn_block, nblocks=nblocks),
        grid=(nblocks,),
        in_specs=[
            pl.BlockSpec((_T, _BN, _F_IN), lambda i: (0, i, 0)),
            pl.BlockSpec((1, 1, _BN), lambda i: (i, 0, 0)),
            full(W1k0), full(W1k1), full(W1k2), full(B1), full(Wa), full(ba),
            full(W2), full(B2), full(W3), full(B3), full(Wb), full(bb),
            full(W4), full(B4), full(lin_w), full(lb),
        ],
        out_specs=pl.BlockSpec((1, lin_w.shape[1]), lambda i: (0, 0)),
        out_shape=jax.ShapeDtypeStruct((1, lin_w.shape[1]), jnp.float32),
        scratch_shapes=[pltpu.VMEM((1, 13 * _BN), jnp.float32)],
    )(x, mask, W1k0, W1k1, W1k2, B1, Wa, ba, W2, B2, W3, B3, Wb, bb, W4, B4,
      lin_w, lb)
    return out[0]


# speedup vs baseline: 1.0286x; 1.0286x over previous
import functools

import jax
import jax.numpy as jnp
from jax.experimental import pallas as pl
from jax.experimental.pallas import tpu as pltpu

_N = 10000
_T = 21
_F_IN = 128
_HID = 32
_BN = 768
_SCALE = 1.0 / (1.0 + 1e-5) ** 0.5


def _pack_taps_t(p):
    w1, b1, w2, b2, w3, b3 = p
    taps = [
        jnp.concatenate([w1[:, :, 0, k], w2[:, :, 0, k], w3[:, :, 0, k]], axis=0)
        for k in range(3)
    ]
    b = jnp.concatenate([b1, b2, b3]).reshape(3 * _HID, 1)
    return taps, b


def _pack_stacked_t(p):
    taps, b = _pack_taps_t(p)
    return jnp.concatenate(taps, axis=1), b


def _glu_t(Y):
    P = Y[0:32, :]
    Q = Y[32:64, :]
    R = Y[64:96, :]
    return jax.nn.relu(P * jax.nn.sigmoid(Q) + R)


def _tap_stack(H, t_out):
    L = t_out * _BN
    return jnp.concatenate(
        [H[:, 0:L], H[:, _BN:_BN + L], H[:, 2 * _BN:2 * _BN + L]], axis=0)


def _stgcn_block(x_ref, mask_ref, w1k0_ref, w1k1_ref, w1k2_ref, b1_ref,
                 wa_ref, ba_ref, w2_ref, b2_ref, w3_ref, b3_ref, wb_ref,
                 bb_ref, w4_ref, b4_ref, lw_ref, lb_ref, out_ref, acc_ref,
                 *, nblocks):
    i = pl.program_id(0)

    xb = x_ref[...].astype(jnp.bfloat16)
    X3 = jnp.transpose(xb, (0, 2, 1))
    xT = jnp.concatenate([X3[t] for t in range(_T)], axis=1)

    dot = functools.partial(jnp.dot, preferred_element_type=jnp.float32)
    A0 = dot(w1k0_ref[...], xT)
    A1 = dot(w1k1_ref[...], xT)
    A2 = dot(w1k2_ref[...], xT)
    L1 = 19 * _BN
    Y1 = (A0[:, 0:L1] + A1[:, _BN:_BN + L1] + A2[:, 2 * _BN:2 * _BN + L1]
          + b1_ref[...])
    H1 = _glu_t(Y1)
    Tc = jax.nn.relu(dot(wa_ref[...], H1) + ba_ref[...])
    H2 = _glu_t(dot(w2_ref[...], _tap_stack(Tc, 17)) + b2_ref[...]) * _SCALE
    H3 = _glu_t(dot(w3_ref[...], _tap_stack(H2, 15)) + b3_ref[...])
    Tc2 = jax.nn.relu(dot(wb_ref[...], H3) + bb_ref[...])
    H4 = _glu_t(dot(w4_ref[...], _tap_stack(Tc2, 13)) + b4_ref[...])

    mask = jnp.concatenate([mask_ref[0]] * 13, axis=1)
    H4 = jnp.where(mask > 0, H4, 0.0)
    part = jnp.sum(H4, axis=0, keepdims=True)

    @pl.when(i == 0)
    def _init():
        acc_ref[...] = jnp.zeros_like(acc_ref)

    acc_ref[...] += part

    @pl.when(i == nblocks - 1)
    def _finish():
        acc = acc_ref[...]
        a13 = jnp.concatenate(
            [acc[:, t * _BN:(t + 1) * _BN] for t in range(13)], axis=0)
        s = jnp.sum(a13, axis=1, keepdims=True)
        out = jnp.sum(s * lw_ref[...], axis=0, keepdims=True)
        out_ref[...] = out * (_SCALE / (_N * _HID)) + lb_ref[...]


def kernel(x, edge_index, edge_weight, tc1a, cheb_a, tc2a, tc1b, cheb_b, tc2b,
           lin_w, lin_b):
    del edge_index, edge_weight
    (W1k0, W1k1, W1k2), B1 = _pack_taps_t(tc1a)
    W1k0, W1k1, W1k2 = (w.astype(jnp.bfloat16) for w in (W1k0, W1k1, W1k2))
    W2, B2 = _pack_stacked_t(tc2a)
    W3, B3 = _pack_stacked_t(tc1b)
    W4, B4 = _pack_stacked_t(tc2b)
    Wa, ba = cheb_a
    Wb, bb = cheb_b
    Wa, Wb = Wa.T, Wb.T
    ba = ba.reshape(_HID, 1)
    bb = bb.reshape(_HID, 1)
    lb = lin_b.reshape(1, -1)

    nblocks = -(-_N // _BN)
    mask = (jnp.arange(nblocks * _BN, dtype=jnp.int32) < _N)
    mask = mask.astype(jnp.float32).reshape(nblocks, 1, _BN)

    full = lambda a: pl.BlockSpec(a.shape, lambda *_: tuple(0 for _ in a.shape))
    out = pl.pallas_call(
        functools.partial(_stgcn_block, nblocks=nblocks),
        grid=(nblocks,),
        in_specs=[
            pl.BlockSpec((_T, _BN, _F_IN), lambda i: (0, i, 0)),
            pl.BlockSpec((1, 1, _BN), lambda i: (i, 0, 0)),
            full(W1k0), full(W1k1), full(W1k2), full(B1), full(Wa), full(ba),
            full(W2), full(B2), full(W3), full(B3), full(Wb), full(bb),
            full(W4), full(B4), full(lin_w), full(lb),
        ],
        out_specs=pl.BlockSpec((1, lin_w.shape[1]), lambda i: (0, 0)),
        out_shape=jax.ShapeDtypeStruct((1, lin_w.shape[1]), jnp.float32),
        scratch_shapes=[pltpu.VMEM((1, 13 * _BN), jnp.float32)],
    )(x, mask, W1k0, W1k1, W1k2, B1, Wa, ba, W2, B2, W3, B3, Wb, bb, W4, B4,
      lin_w, lb)
    return out[0]


# speedup vs baseline: 1.0799x; 1.0499x over previous
import functools

import jax
import jax.numpy as jnp
import numpy as np
from jax.experimental import pallas as pl
from jax.experimental.pallas import tpu as pltpu

_N = 10000
_T = 21
_F_IN = 128
_HID = 32
_BN = 768
_SCALE = 1.0 / (1.0 + 1e-5) ** 0.5


def _pack_taps_t(p):
    w1, b1, w2, b2, w3, b3 = p
    taps = [
        jnp.concatenate([w1[:, :, 0, k], w2[:, :, 0, k], w3[:, :, 0, k]], axis=0)
        for k in range(3)
    ]
    b = jnp.concatenate([b1, b2, b3]).reshape(3 * _HID, 1)
    return taps, b


def _pack_stacked_t(p):
    taps, b = _pack_taps_t(p)
    return jnp.concatenate(taps, axis=1), b


def _glu_t(Y):
    P = Y[0:32, :]
    Q = Y[32:64, :]
    R = Y[64:96, :]
    return jax.nn.relu(P * jax.nn.sigmoid(Q) + R)


def _tap_stack(H, t_out):
    L = t_out * _BN
    return jnp.concatenate(
        [H[:, 0:L], H[:, _BN:_BN + L], H[:, 2 * _BN:2 * _BN + L]], axis=0)


def _stgcn_block(x_ref, mask_ref, w1_ref, w234_ref, wab_ref, b14_ref,
                 bab_ref, lw_ref, lb_ref, out_ref, acc_ref, *, nblocks):
    i = pl.program_id(0)

    xb = x_ref[...].astype(jnp.bfloat16)
    X3 = jnp.transpose(xb, (0, 2, 1))
    xT = jnp.concatenate([X3[t] for t in range(_T)], axis=1)

    dot = functools.partial(jnp.dot, preferred_element_type=jnp.float32)
    A0 = dot(w1_ref[0], xT)
    A1 = dot(w1_ref[1], xT)
    A2 = dot(w1_ref[2], xT)
    L1 = 19 * _BN
    Y1 = (A0[:, 0:L1] + A1[:, _BN:_BN + L1] + A2[:, 2 * _BN:2 * _BN + L1]
          + b14_ref[0])
    H1 = _glu_t(Y1)
    Tc = jax.nn.relu(dot(wab_ref[0], H1) + bab_ref[0])
    H2 = _glu_t(dot(w234_ref[0], _tap_stack(Tc, 17)) + b14_ref[1]) * _SCALE
    H3 = _glu_t(dot(w234_ref[1], _tap_stack(H2, 15)) + b14_ref[2])
    Tc2 = jax.nn.relu(dot(wab_ref[1], H3) + bab_ref[1])
    H4 = _glu_t(dot(w234_ref[2], _tap_stack(Tc2, 13)) + b14_ref[3])

    mask = jnp.concatenate([mask_ref[0]] * 13, axis=1)
    H4 = jnp.where(mask > 0, H4, 0.0)
    part = jnp.sum(H4, axis=0, keepdims=True)

    @pl.when(i == 0)
    def _init():
        acc_ref[...] = jnp.zeros_like(acc_ref)

    acc_ref[...] += part

    @pl.when(i == nblocks - 1)
    def _finish():
        acc = acc_ref[...]
        a13 = jnp.concatenate(
            [acc[:, t * _BN:(t + 1) * _BN] for t in range(13)], axis=0)
        s = jnp.sum(a13, axis=1, keepdims=True)
        out = jnp.sum(s * lw_ref[...], axis=0, keepdims=True)
        out_ref[...] = out * (_SCALE / (_N * _HID)) + lb_ref[...]


def kernel(x, edge_index, edge_weight, tc1a, cheb_a, tc2a, tc1b, cheb_b, tc2b,
           lin_w, lin_b):
    del edge_index, edge_weight
    taps1, B1 = _pack_taps_t(tc1a)
    W1 = jnp.stack(taps1).astype(jnp.bfloat16)
    W2, B2 = _pack_stacked_t(tc2a)
    W3, B3 = _pack_stacked_t(tc1b)
    W4, B4 = _pack_stacked_t(tc2b)
    W234 = jnp.stack([W2, W3, W4])
    B14 = jnp.stack([B1, B2, B3, B4])
    Wa, ba = cheb_a
    Wb, bb = cheb_b
    Wab = jnp.stack([Wa.T, Wb.T])
    Bab = jnp.stack([ba.reshape(_HID, 1), bb.reshape(_HID, 1)])
    lb = lin_b.reshape(1, -1)

    nblocks = -(-_N // _BN)
    mask = np.arange(nblocks * _BN) < _N
    mask = jnp.asarray(mask.astype(np.float32).reshape(nblocks, 1, _BN))

    full = lambda a: pl.BlockSpec(a.shape, lambda *_: tuple(0 for _ in a.shape))
    out = pl.pallas_call(
        functools.partial(_stgcn_block, nblocks=nblocks),
        grid=(nblocks,),
        in_specs=[
            pl.BlockSpec((_T, _BN, _F_IN), lambda i: (0, i, 0)),
            pl.BlockSpec((1, 1, _BN), lambda i: (i, 0, 0)),
            full(W1), full(W234), full(Wab), full(B14), full(Bab),
            full(lin_w), full(lb),
        ],
        out_specs=pl.BlockSpec((1, lin_w.shape[1]), lambda i: (0, 0)),
        out_shape=jax.ShapeDtypeStruct((1, lin_w.shape[1]), jnp.float32),
        scratch_shapes=[pltpu.VMEM((1, 13 * _BN), jnp.float32)],
    )(x, mask, W1, W234, Wab, B14, Bab, lin_w, lb)
    return out[0]


# speedup vs baseline: 1.1364x; 1.0523x over previous
import functools

import jax
import jax.numpy as jnp
import numpy as np
from jax.experimental import pallas as pl
from jax.experimental.pallas import tpu as pltpu

_N = 10000
_T = 21
_F_IN = 128
_HID = 32
_BN = 768
_SCALE = 1.0 / (1.0 + 1e-5) ** 0.5


def _pack_taps_t(p):
    w1, b1, w2, b2, w3, b3 = p
    taps = [
        jnp.concatenate([w1[:, :, 0, k], w2[:, :, 0, k], w3[:, :, 0, k]], axis=0)
        for k in range(3)
    ]
    b = jnp.concatenate([b1, b2, b3]).reshape(3 * _HID, 1)
    return taps, b


def _pack_stacked_t(p):
    taps, b = _pack_taps_t(p)
    return jnp.concatenate(taps, axis=1), b


def _glu_t(Y):
    P = Y[0:32, :]
    Q = Y[32:64, :]
    R = Y[64:96, :]
    return jax.nn.relu(P * jax.nn.sigmoid(Q) + R)


def _tap_stack(H, t_out):
    L = t_out * _BN
    return jnp.concatenate(
        [H[:, 0:L], H[:, _BN:_BN + L], H[:, 2 * _BN:2 * _BN + L]], axis=0)


def _stgcn_block(x_ref, mask_ref, w1_ref, w234_ref, wab_ref, b14_ref,
                 bab_ref, lw_ref, lb_ref, out_ref, acc_ref, *, nblocks):
    i = pl.program_id(0)

    xb = x_ref[...].astype(jnp.bfloat16)
    X3 = jnp.transpose(xb, (0, 2, 1))
    xT = jnp.concatenate([X3[t] for t in range(_T)], axis=1)

    dot = functools.partial(jnp.dot, preferred_element_type=jnp.float32)
    A0 = dot(w1_ref[0], xT)
    A1 = dot(w1_ref[1], xT)
    A2 = dot(w1_ref[2], xT)
    L1 = 19 * _BN
    Y1 = (A0[:, 0:L1] + A1[:, _BN:_BN + L1] + A2[:, 2 * _BN:2 * _BN + L1]
          + b14_ref[0])
    H1 = _glu_t(Y1)
    Tc = jax.nn.relu(dot(wab_ref[0], H1) + bab_ref[0])
    H2 = _glu_t(dot(w234_ref[0], _tap_stack(Tc, 17)) + b14_ref[1]) * _SCALE
    H3 = _glu_t(dot(w234_ref[1], _tap_stack(H2, 15)) + b14_ref[2])
    Tc2 = jax.nn.relu(dot(wab_ref[1], H3) + bab_ref[1])
    H4 = _glu_t(dot(w234_ref[2], _tap_stack(Tc2, 13)) + b14_ref[3])

    mask = jnp.concatenate([mask_ref[0]] * 13, axis=1)
    H4 = jnp.where(mask > 0, H4, 0.0)
    part = jnp.sum(H4, axis=0, keepdims=True)

    @pl.when(i == 0)
    def _init():
        acc_ref[...] = jnp.zeros_like(acc_ref)

    acc_ref[...] += part

    @pl.when(i == nblocks - 1)
    def _finish():
        acc = acc_ref[...]
        a13 = jnp.concatenate(
            [acc[:, t * _BN:(t + 1) * _BN] for t in range(13)], axis=0)
        s = jnp.sum(a13, axis=1, keepdims=True)
        out = jnp.sum(s * lw_ref[...], axis=0, keepdims=True)
        out_ref[...] = out * (_SCALE / (_N * _HID)) + lb_ref[...]


def kernel(x, edge_index, edge_weight, tc1a, cheb_a, tc2a, tc1b, cheb_b, tc2b,
           lin_w, lin_b):
    del edge_index, edge_weight
    cat1 = jnp.concatenate([tc1a[0], tc1a[2], tc1a[4]], axis=0)
    W1 = jnp.transpose(cat1.reshape(96, _F_IN, 3), (2, 0, 1)).astype(jnp.bfloat16)
    cat234 = jnp.concatenate(
        [tc2a[0], tc2a[2], tc2a[4], tc1b[0], tc1b[2], tc1b[4],
         tc2b[0], tc2b[2], tc2b[4]], axis=0)
    W234 = jnp.transpose(cat234.reshape(288, _HID, 3), (0, 2, 1)).reshape(3, 96, 96)
    B14 = jnp.concatenate(
        [tc1a[1], tc1a[3], tc1a[5], tc2a[1], tc2a[3], tc2a[5],
         tc1b[1], tc1b[3], tc1b[5], tc2b[1], tc2b[3], tc2b[5]]).reshape(4, 96, 1)
    Wab = jnp.transpose(jnp.stack([cheb_a[0], cheb_b[0]]), (0, 2, 1))
    Bab = jnp.concatenate([cheb_a[1], cheb_b[1]]).reshape(2, _HID, 1)
    lb = lin_b.reshape(1, -1)

    nblocks = -(-_N // _BN)
    mask = np.arange(nblocks * _BN) < _N
    mask = jnp.asarray(mask.astype(np.float32).reshape(nblocks, 1, _BN))

    full = lambda a: pl.BlockSpec(a.shape, lambda *_: tuple(0 for _ in a.shape))
    out = pl.pallas_call(
        functools.partial(_stgcn_block, nblocks=nblocks),
        grid=(nblocks,),
        in_specs=[
            pl.BlockSpec((_T, _BN, _F_IN), lambda i: (0, i, 0)),
            pl.BlockSpec((1, 1, _BN), lambda i: (i, 0, 0)),
            full(W1), full(W234), full(Wab), full(B14), full(Bab),
            full(lin_w), full(lb),
        ],
        out_specs=pl.BlockSpec((1, lin_w.shape[1]), lambda i: (0, 0)),
        out_shape=jax.ShapeDtypeStruct((1, lin_w.shape[1]), jnp.float32),
        scratch_shapes=[pltpu.VMEM((1, 13 * _BN), jnp.float32)],
    )(x, mask, W1, W234, Wab, B14, Bab, lin_w, lb)
    return out[0]


# speedup vs baseline: 1.1451x; 1.0077x over previous
import functools

import jax
import jax.numpy as jnp
import numpy as np
from jax.experimental import pallas as pl
from jax.experimental.pallas import tpu as pltpu

_N = 10000
_T = 21
_F_IN = 128
_HID = 32
_BN = 768
_SCALE = 1.0 / (1.0 + 1e-5) ** 0.5


def _pack_taps_t(p):
    w1, b1, w2, b2, w3, b3 = p
    taps = [
        jnp.concatenate([w1[:, :, 0, k], w2[:, :, 0, k], w3[:, :, 0, k]], axis=0)
        for k in range(3)
    ]
    b = jnp.concatenate([b1, b2, b3]).reshape(3 * _HID, 1)
    return taps, b


def _pack_stacked_t(p):
    taps, b = _pack_taps_t(p)
    return jnp.concatenate(taps, axis=1), b


def _glu_t(Y):
    P = Y[0:32, :]
    Q = Y[32:64, :]
    R = Y[64:96, :]
    return jax.nn.relu(P * jax.nn.sigmoid(Q) + R)


def _tap_stack(H, t_out):
    L = t_out * _BN
    return jnp.concatenate(
        [H[:, 0:L], H[:, _BN:_BN + L], H[:, 2 * _BN:2 * _BN + L]], axis=0)


def _stgcn_block(x_ref, mask_ref, w1_ref, w234_ref, wab_ref, b14_ref,
                 bab_ref, lw_ref, lb_ref, out_ref, acc_ref, *, nblocks):
    i = pl.program_id(0)

    xb = x_ref[...].astype(jnp.bfloat16)
    X3 = jnp.transpose(xb, (0, 2, 1))
    xT = jnp.concatenate([X3[t] for t in range(_T)], axis=1)

    dot = functools.partial(jnp.dot, preferred_element_type=jnp.float32)
    A0 = dot(w1_ref[0], xT)
    A1 = dot(w1_ref[1], xT)
    A2 = dot(w1_ref[2], xT)
    L1 = 19 * _BN
    Y1 = (A0[:, 0:L1] + A1[:, _BN:_BN + L1] + A2[:, 2 * _BN:2 * _BN + L1]
          + b14_ref[0])
    H1 = _glu_t(Y1)
    Tc = jax.nn.relu(dot(wab_ref[0], H1) + bab_ref[0])
    H2 = _glu_t(dot(w234_ref[0], _tap_stack(Tc, 17)) + b14_ref[1]) * _SCALE
    H3 = _glu_t(dot(w234_ref[1], _tap_stack(H2, 15)) + b14_ref[2])
    Tc2 = jax.nn.relu(dot(wab_ref[1], H3) + bab_ref[1])
    H4 = _glu_t(dot(w234_ref[2], _tap_stack(Tc2, 13)) + b14_ref[3])

    mask = jnp.concatenate([mask_ref[0]] * 13, axis=1)
    H4 = jnp.where(mask > 0, H4, 0.0)
    part = jnp.sum(H4, axis=0, keepdims=True)

    @pl.when(i == 0)
    def _init():
        acc_ref[...] = jnp.zeros_like(acc_ref)

    acc_ref[...] += part

    @pl.when(i == nblocks - 1)
    def _finish():
        acc = acc_ref[...]
        a13 = jnp.concatenate(
            [acc[:, t * _BN:(t + 1) * _BN] for t in range(13)], axis=0)
        s = jnp.sum(a13, axis=1, keepdims=True)
        out = jnp.sum(s * lw_ref[...], axis=0, keepdims=True)
        out_ref[...] = out * (_SCALE / (_N * _HID)) + lb_ref[...]


def kernel(x, edge_index, edge_weight, tc1a, cheb_a, tc2a, tc1b, cheb_b, tc2b,
           lin_w, lin_b):
    del edge_index, edge_weight
    cat1 = jnp.concatenate(
        [tc1a[0].reshape(_HID, -1), tc1a[2].reshape(_HID, -1),
         tc1a[4].reshape(_HID, -1)], axis=0)
    W1 = jnp.transpose(cat1.reshape(96, _F_IN, 3), (2, 0, 1)).astype(jnp.bfloat16)
    cat234 = jnp.concatenate(
        [w.reshape(_HID, -1) for w in
         (tc2a[0], tc2a[2], tc2a[4], tc1b[0], tc1b[2], tc1b[4],
          tc2b[0], tc2b[2], tc2b[4])], axis=0)
    W234 = jnp.transpose(cat234.reshape(288, _HID, 3), (0, 2, 1)).reshape(3, 96, 96)
    B14 = jnp.concatenate(
        [tc1a[1], tc1a[3], tc1a[5], tc2a[1], tc2a[3], tc2a[5],
         tc1b[1], tc1b[3], tc1b[5], tc2b[1], tc2b[3], tc2b[5]]).reshape(4, 96, 1)
    Wab = jnp.transpose(jnp.stack([cheb_a[0], cheb_b[0]]), (0, 2, 1))
    Bab = jnp.concatenate([cheb_a[1], cheb_b[1]]).reshape(2, _HID, 1)
    lb = lin_b.reshape(1, -1)

    nblocks = -(-_N // _BN)
    mask = np.arange(nblocks * _BN) < _N
    mask = jnp.asarray(mask.astype(np.float32).reshape(nblocks, 1, _BN))

    full = lambda a: pl.BlockSpec(a.shape, lambda *_: tuple(0 for _ in a.shape))
    out = pl.pallas_call(
        functools.partial(_stgcn_block, nblocks=nblocks),
        grid=(nblocks,),
        in_specs=[
            pl.BlockSpec((_T, _BN, _F_IN), lambda i: (0, i, 0)),
            pl.BlockSpec((1, 1, _BN), lambda i: (i, 0, 0)),
            full(W1), full(W234), full(Wab), full(B14), full(Bab),
            full(lin_w), full(lb),
        ],
        out_specs=pl.BlockSpec((1, lin_w.shape[1]), lambda i: (0, 0)),
        out_shape=jax.ShapeDtypeStruct((1, lin_w.shape[1]), jnp.float32),
        scratch_shapes=[pltpu.VMEM((1, 13 * _BN), jnp.float32)],
    )(x, mask, W1, W234, Wab, B14, Bab, lin_w, lb)
    return out[0]
